# baseline (speedup 1.0000x reference)
import numpy as np
import jax
import jax.numpy as jnp
from jax import lax
from jax.experimental import pallas as pl
from jax.experimental.pallas import tpu as pltpu
from jax.experimental.pallas import tpu_sc as plsc

N_NODES = 10000
N_EDGES = 160000
K = 16
HID = 128
BLK = 1000

NT = 16
EPT = 10240
E_PAD = NT * EPT
CH = 128
NCH = EPT // CH
NB = 2
RPT = 624
TAIL = N_NODES - NT * RPT
TRASH = N_NODES


def _sc_body(feats, rec, zeros, accs_out,
             rec0, rec1, idxb0, idxb1, bas0, bas1, rows0, rows1, acc_sh,
             semr0, semr1, semg0, semg1, semsc0, semsc1):
    c = lax.axis_index("c")
    s = lax.axis_index("s")
    f32 = jnp.float32
    recs = (rec0, rec1)
    idxbs = (idxb0, idxb1)
    bass = (bas0, bas1)
    rowss = (rows0, rows1)
    semrs = (semr0, semr1)
    semgs = (semg0, semg1)
    semscs = (semsc0, semsc1)

    def _zero_acc():
        pltpu.sync_copy(zeros.at[pl.ds(0, RPT), :],
                        acc_sh.at[pl.ds(s * RPT, RPT), :])

        @pl.when(s == 0)
        def _():
            pltpu.sync_copy(zeros.at[pl.ds(0, TAIL + 8), :],
                            acc_sh.at[pl.ds(NT * RPT, TAIL + 8), :])

    def _writeback(task):
        pltpu.sync_copy(acc_sh.at[pl.ds(s * RPT, RPT), :],
                        accs_out.at[task, pl.ds(s * RPT, RPT), :])

        @pl.when(s == 0)
        def _():
            pltpu.sync_copy(acc_sh.at[pl.ds(NT * RPT, TAIL), :],
                            accs_out.at[task, pl.ds(NT * RPT, TAIL), :])

    def _basis(p, rb, ba):
        for g in range(CH // 16):
            b = jnp.ones((16,), f32)
            for d in range(4):
                bf = ((p >> d) & 1).astype(f32)
                td = lax.bitcast_convert_type(rb[2 + d, pl.ds(g * 16, 16)],
                                              f32)
                b = b * ((1.0 - td) + bf * (2.0 * td - 1.0))
            ba[pl.ds(g * 16, 16)] = b

    def _scale(ba, rw):
        @plsc.parallel_loop(0, CH, step=1, unroll=4)
        def scale(e):
            bv = ba[pl.ds(e, 16)][0]
            for u in range(8):
                sl = pl.ds(u * 16, 16)
                rw[e, sl] = rw[e, sl] * bv

    def _drain_scatter(b):
        pltpu.make_async_copy(zeros.at[pl.ds(0, CH), :], rowss[b],
                              semscs[b]).wait()

    def one_pass(p, _):
        _zero_acc()
        plsc.subcore_barrier()

        def batch(q, _):
            drs = []
            for b in range(NB):
                j = q * NB + b

                @pl.when(q > 0)
                def _():
                    _drain_scatter(b)
                drs.append(pltpu.async_copy(rec.at[s, j], recs[b], semrs[b]))
            for b in range(NB):
                drs[b].wait()
                for v in range(CH // 16):
                    sl = pl.ds(v * 16, 16)
                    idxbs[b][sl] = recs[b][0, sl] + c * N_NODES
                dg = pltpu.async_copy(feats.at[idxbs[b]], rowss[b], semgs[b])
                _basis(p, recs[b], bass[b])
                dg.wait()
                _scale(bass[b], rowss[b])
                pltpu.async_copy(rowss[b], acc_sh.at[recs[b].at[1]],
                                 semscs[b], add=True)
            return 0
        lax.fori_loop(0, NCH // NB, batch, 0)
        for b in range(NB):
            _drain_scatter(b)

        plsc.subcore_barrier()
        _writeback(c * K + p)
        return 0

    lax.fori_loop(0, K, one_pass, 0)

    _zero_acc()
    plsc.subcore_barrier()

    def _fill_ones(r, _):
        for u in range(8):
            rows0[r, pl.ds(u * 16, 16)] = jnp.ones((16,), f32)
        return 0
    lax.fori_loop(0, CH, _fill_ones, 0)

    def dchunk(j, _):
        pltpu.async_copy(rec.at[s, j], rec0, semr0).wait()
        pltpu.sync_copy(rows0, acc_sh.at[rec0.at[1]], add=True)
        return 0
    lax.fori_loop(0, NCH, dchunk, 0)
    plsc.subcore_barrier()

    @pl.when(c == 0)
    def _():
        _writeback(2 * K)


_sc_scatter = pl.kernel(
    _sc_body,
    mesh=plsc.VectorSubcoreMesh(core_axis_name="c", subcore_axis_name="s"),
    out_type=jax.ShapeDtypeStruct((2 * K + 1, N_NODES, HID), jnp.float32),
    scratch_types=[
        pltpu.VMEM((6, CH), jnp.int32),
        pltpu.VMEM((6, CH), jnp.int32),
        pltpu.VMEM((CH,), jnp.int32),
        pltpu.VMEM((CH,), jnp.int32),
        pltpu.VMEM((CH + 16,), jnp.float32),
        pltpu.VMEM((CH + 16,), jnp.float32),
        pltpu.VMEM((CH, HID), jnp.float32),
        pltpu.VMEM((CH, HID), jnp.float32),
        pltpu.VMEM_SHARED((N_NODES + 8, HID), jnp.float32),
        pltpu.SemaphoreType.DMA,
        pltpu.SemaphoreType.DMA,
        pltpu.SemaphoreType.DMA,
        pltpu.SemaphoreType.DMA,
        pltpu.SemaphoreType.DMA,
        pltpu.SemaphoreType.DMA,
    ],
)


def _dense_body(accx_ref, acch_ref, deg_ref, x_ref, h_ref,
                wx_ref, wh_ref, rx_ref, rh_ref, bx_ref, bh_ref, out_ref):
    f32 = jnp.float32
    dn = (((1,), (0,)), ((), ()))
    ax = lax.dot_general(x_ref[...], rx_ref[...], dn, preferred_element_type=f32)
    ah = lax.dot_general(h_ref[...], rh_ref[...], dn, preferred_element_type=f32)
    axs = jnp.zeros_like(ax)
    ahs = jnp.zeros_like(ah)
    for k in range(K):
        axs += lax.dot_general(accx_ref[k], wx_ref[k], dn, preferred_element_type=f32)
        ahs += lax.dot_general(acch_ref[k], wh_ref[k], dn, preferred_element_type=f32)
    dinv = 1.0 / jnp.maximum(deg_ref[0][:, 0:1], 1.0)
    ax = ax + axs * dinv + bx_ref[...]
    ah = ah + ahs * dinv + bh_ref[...]
    xr_o = ax[:, 0:128]
    xz_o = ax[:, 128:256]
    xn_o = ax[:, 256:384]
    hr_o = ah[:, 0:128]
    hz_o = ah[:, 128:256]
    r = jax.nn.sigmoid(xr_o + hr_o)
    z = jax.nn.sigmoid(xz_o + hz_o)
    n = jnp.tanh(xn_o + r * hr_o)
    out_ref[...] = (1.0 - z) * n + z * h_ref[...]


def _dense_phase(accs, x, hidden, wx, wh, rx, rh, bx, bh):
    grid = (N_NODES // BLK,)
    return pl.pallas_call(
        _dense_body,
        grid=grid,
        in_specs=[
            pl.BlockSpec((K, BLK, HID), lambda i: (0, i, 0)),
            pl.BlockSpec((K, BLK, HID), lambda i: (1, i, 0)),
            pl.BlockSpec((1, BLK, HID), lambda i: (2 * K, i, 0)),
            pl.BlockSpec((BLK, HID), lambda i: (i, 0)),
            pl.BlockSpec((BLK, HID), lambda i: (i, 0)),
            pl.BlockSpec((K, HID, 3 * HID), lambda i: (0, 0, 0)),
            pl.BlockSpec((K, HID, 2 * HID), lambda i: (0, 0, 0)),
            pl.BlockSpec((HID, 3 * HID), lambda i: (0, 0)),
            pl.BlockSpec((HID, 2 * HID), lambda i: (0, 0)),
            pl.BlockSpec((1, 3 * HID), lambda i: (0, 0)),
            pl.BlockSpec((1, 2 * HID), lambda i: (0, 0)),
        ],
        out_specs=pl.BlockSpec((BLK, HID), lambda i: (i, 0)),
        out_shape=jax.ShapeDtypeStruct((N_NODES, HID), jnp.float32),
    )(accs, accs, accs, x, hidden, wx, wh, rx, rh, bx, bh)


def kernel(x, hidden, edge_index, edge_attr,
           xr_w, xr_root, xr_b, hr_w, hr_root, hr_b,
           xz_w, xz_root, xz_b, hz_w, hz_root, hz_b,
           xn_w, xn_root, xn_b):
    src = edge_index[0].astype(jnp.int32)
    dst = edge_index[1].astype(jnp.int32)
    npad = E_PAD - N_EDGES
    srcp = jnp.concatenate([src, jnp.zeros((npad,), jnp.int32)])
    dstp = jnp.concatenate([dst, jnp.full((npad,), TRASH, jnp.int32)])
    eatp = jnp.concatenate(
        [edge_attr, jnp.full((npad, 4), 0.5, jnp.float32)])
    feats = jnp.concatenate([x, hidden], axis=0)
    rec = jnp.stack([srcp, dstp] +
                    [lax.bitcast_convert_type(eatp[:, d], jnp.int32)
                     for d in range(4)])
    rec = rec.reshape(6, NT, NCH, CH).transpose(1, 2, 0, 3)
    zeros = jnp.zeros((RPT + TAIL + 8, HID), jnp.float32)

    accs = _sc_scatter(feats, rec, zeros)

    wx = jnp.concatenate([xr_w, xz_w, xn_w], axis=2)
    wh = jnp.concatenate([hr_w, hz_w], axis=2)
    rx = jnp.concatenate([xr_root, xz_root, xn_root], axis=1)
    rh = jnp.concatenate([hr_root, hz_root], axis=1)
    bx = jnp.concatenate([xr_b, xz_b, xn_b])[None, :]
    bh = jnp.concatenate([hr_b, hz_b])[None, :]
    return _dense_phase(accs, x, hidden, wx, wh, rx, rh, bx, bh)


# speedup vs baseline: 1.3182x; 1.3182x over previous
import numpy as np
import jax
import jax.numpy as jnp
from jax import lax
from jax.experimental import pallas as pl
from jax.experimental.pallas import tpu as pltpu
from jax.experimental.pallas import tpu_sc as plsc

N_NODES = 10000
N_EDGES = 160000
K = 16
HID = 128
BLK = 1000

NT = 16
EPT = 10080
E_PAD = NT * EPT
CH = 80
NCH = EPT // CH
NB = 2
RPT = 624
TAIL = N_NODES - NT * RPT
TRASH = N_NODES


def _sc_body(feats, rec, zeros, accs_out,
             rec0, rec1, idxb0, idxb1, bas0, bas1, rows0, rows1, acc_sh,
             semr0, semr1, semg0, semg1, semsc0, semsc1):
    c = lax.axis_index("c")
    s = lax.axis_index("s")
    f32 = jnp.float32
    recs = (rec0, rec1)
    idxbs = (idxb0, idxb1)
    bass = (bas0, bas1)
    rowss = (rows0, rows1)
    semrs = (semr0, semr1)
    semgs = (semg0, semg1)
    semscs = (semsc0, semsc1)

    def _zero_acc():
        pltpu.sync_copy(zeros.at[pl.ds(0, RPT), :],
                        acc_sh.at[pl.ds(s * RPT, RPT), :])

        @pl.when(s == 0)
        def _():
            pltpu.sync_copy(zeros.at[pl.ds(0, TAIL + 8), :],
                            acc_sh.at[pl.ds(NT * RPT, TAIL + 8), :])

    def _writeback(task):
        pltpu.sync_copy(acc_sh.at[pl.ds(s * RPT, RPT), :],
                        accs_out.at[task, pl.ds(s * RPT, RPT), :])

        @pl.when(s == 0)
        def _():
            pltpu.sync_copy(acc_sh.at[pl.ds(NT * RPT, TAIL), :],
                            accs_out.at[task, pl.ds(NT * RPT, TAIL), :])

    def _basis(p, rb, ba):
        for g in range(CH // 16):
            b = jnp.ones((16,), f32)
            for d in range(4):
                bf = ((p >> d) & 1).astype(f32)
                td = lax.bitcast_convert_type(rb[2 + d, pl.ds(g * 16, 16)],
                                              f32)
                b = b * ((1.0 - td) + bf * (2.0 * td - 1.0))
            ba[pl.ds(g * 16, 16)] = b

    def _scale(ba, rw):
        @plsc.parallel_loop(0, CH, step=1, unroll=4)
        def scale(e):
            bv = ba[pl.ds(e, 16)][0]
            for u in range(8):
                sl = pl.ds(u * 16, 16)
                rw[e, sl] = rw[e, sl] * bv

    def _drain_scatter(b):
        pltpu.make_async_copy(zeros.at[pl.ds(0, CH), :], rowss[b],
                              semscs[b]).wait()

    def one_pass(p, _):
        _zero_acc()
        plsc.subcore_barrier()

        def batch(q, _):
            drs = []
            for b in range(NB):
                j = q * NB + b

                @pl.when(q > 0)
                def _():
                    _drain_scatter(b)
                drs.append(pltpu.async_copy(rec.at[s, j], recs[b], semrs[b]))
            for b in range(NB):
                drs[b].wait()
                for v in range(CH // 16):
                    sl = pl.ds(v * 16, 16)
                    idxbs[b][sl] = recs[b][0, sl] + c * N_NODES
                dg = pltpu.async_copy(feats.at[idxbs[b]], rowss[b], semgs[b])
                _basis(p, recs[b], bass[b])
                dg.wait()
                _scale(bass[b], rowss[b])
                pltpu.async_copy(rowss[b], acc_sh.at[recs[b].at[1]],
                                 semscs[b], add=True)
            return 0
        lax.fori_loop(0, NCH // NB, batch, 0)
        for b in range(NB):
            _drain_scatter(b)

        plsc.subcore_barrier()
        _writeback(c * K + p)
        return 0

    lax.fori_loop(0, K, one_pass, 0)

    _zero_acc()
    plsc.subcore_barrier()

    def _fill_ones(r, _):
        for u in range(8):
            rows0[r, pl.ds(u * 16, 16)] = jnp.ones((16,), f32)
        return 0
    lax.fori_loop(0, CH, _fill_ones, 0)

    def dchunk(j, _):
        pltpu.async_copy(rec.at[s, j], rec0, semr0).wait()
        pltpu.sync_copy(rows0, acc_sh.at[rec0.at[1]], add=True)
        return 0
    lax.fori_loop(0, NCH, dchunk, 0)
    plsc.subcore_barrier()

    @pl.when(c == 0)
    def _():
        _writeback(2 * K)


_sc_scatter = pl.kernel(
    _sc_body,
    mesh=plsc.VectorSubcoreMesh(core_axis_name="c", subcore_axis_name="s"),
    out_type=jax.ShapeDtypeStruct((2 * K + 1, N_NODES, HID), jnp.float32),
    scratch_types=[
        pltpu.VMEM((6, CH), jnp.int32),
        pltpu.VMEM((6, CH), jnp.int32),
        pltpu.VMEM((CH,), jnp.int32),
        pltpu.VMEM((CH,), jnp.int32),
        pltpu.VMEM((CH + 16,), jnp.float32),
        pltpu.VMEM((CH + 16,), jnp.float32),
        pltpu.VMEM((CH, HID), jnp.float32),
        pltpu.VMEM((CH, HID), jnp.float32),
        pltpu.VMEM_SHARED((N_NODES + 8, HID), jnp.float32),
        pltpu.SemaphoreType.DMA,
        pltpu.SemaphoreType.DMA,
        pltpu.SemaphoreType.DMA,
        pltpu.SemaphoreType.DMA,
        pltpu.SemaphoreType.DMA,
        pltpu.SemaphoreType.DMA,
    ],
)


def _dense_body(accx_ref, acch_ref, deg_ref, x_ref, h_ref,
                wx_ref, wh_ref, rx_ref, rh_ref, bx_ref, bh_ref, out_ref):
    f32 = jnp.float32
    dn = (((1,), (0,)), ((), ()))
    ax = lax.dot_general(x_ref[...], rx_ref[...], dn, preferred_element_type=f32)
    ah = lax.dot_general(h_ref[...], rh_ref[...], dn, preferred_element_type=f32)
    axs = jnp.zeros_like(ax)
    ahs = jnp.zeros_like(ah)
    for k in range(K):
        axs += lax.dot_general(accx_ref[k], wx_ref[k], dn, preferred_element_type=f32)
        ahs += lax.dot_general(acch_ref[k], wh_ref[k], dn, preferred_element_type=f32)
    dinv = 1.0 / jnp.maximum(deg_ref[0][:, 0:1], 1.0)
    ax = ax + axs * dinv + bx_ref[...]
    ah = ah + ahs * dinv + bh_ref[...]
    xr_o = ax[:, 0:128]
    xz_o = ax[:, 128:256]
    xn_o = ax[:, 256:384]
    hr_o = ah[:, 0:128]
    hz_o = ah[:, 128:256]
    r = jax.nn.sigmoid(xr_o + hr_o)
    z = jax.nn.sigmoid(xz_o + hz_o)
    n = jnp.tanh(xn_o + r * hr_o)
    out_ref[...] = (1.0 - z) * n + z * h_ref[...]


def _dense_phase(accs, x, hidden, wx, wh, rx, rh, bx, bh):
    grid = (N_NODES // BLK,)
    return pl.pallas_call(
        _dense_body,
        grid=grid,
        in_specs=[
            pl.BlockSpec((K, BLK, HID), lambda i: (0, i, 0)),
            pl.BlockSpec((K, BLK, HID), lambda i: (1, i, 0)),
            pl.BlockSpec((1, BLK, HID), lambda i: (2 * K, i, 0)),
            pl.BlockSpec((BLK, HID), lambda i: (i, 0)),
            pl.BlockSpec((BLK, HID), lambda i: (i, 0)),
            pl.BlockSpec((K, HID, 3 * HID), lambda i: (0, 0, 0)),
            pl.BlockSpec((K, HID, 2 * HID), lambda i: (0, 0, 0)),
            pl.BlockSpec((HID, 3 * HID), lambda i: (0, 0)),
            pl.BlockSpec((HID, 2 * HID), lambda i: (0, 0)),
            pl.BlockSpec((1, 3 * HID), lambda i: (0, 0)),
            pl.BlockSpec((1, 2 * HID), lambda i: (0, 0)),
        ],
        out_specs=pl.BlockSpec((BLK, HID), lambda i: (i, 0)),
        out_shape=jax.ShapeDtypeStruct((N_NODES, HID), jnp.float32),
    )(accs, accs, accs, x, hidden, wx, wh, rx, rh, bx, bh)


def kernel(x, hidden, edge_index, edge_attr,
           xr_w, xr_root, xr_b, hr_w, hr_root, hr_b,
           xz_w, xz_root, xz_b, hz_w, hz_root, hz_b,
           xn_w, xn_root, xn_b):
    src = edge_index[0].astype(jnp.int32)
    dst = edge_index[1].astype(jnp.int32)
    npad = E_PAD - N_EDGES
    srcp = jnp.concatenate([src, jnp.zeros((npad,), jnp.int32)])
    dstp = jnp.concatenate([dst, jnp.full((npad,), TRASH, jnp.int32)])
    eatp = jnp.concatenate(
        [edge_attr, jnp.full((npad, 4), 0.5, jnp.float32)])
    feats = jnp.concatenate([x, hidden], axis=0)
    rec = jnp.stack([srcp, dstp] +
                    [lax.bitcast_convert_type(eatp[:, d], jnp.int32)
                     for d in range(4)])
    rec = rec.reshape(6, NT, NCH, CH).transpose(1, 2, 0, 3)
    zeros = jnp.zeros((RPT + TAIL + 8, HID), jnp.float32)

    accs = _sc_scatter(feats, rec, zeros)

    wx = jnp.concatenate([xr_w, xz_w, xn_w], axis=2)
    wh = jnp.concatenate([hr_w, hz_w], axis=2)
    rx = jnp.concatenate([xr_root, xz_root, xn_root], axis=1)
    rh = jnp.concatenate([hr_root, hz_root], axis=1)
    bx = jnp.concatenate([xr_b, xz_b, xn_b])[None, :]
    bh = jnp.concatenate([hr_b, hz_b])[None, :]
    return _dense_phase(accs, x, hidden, wx, wh, rx, rh, bx, bh)


# speedup vs baseline: 1.6938x; 1.2849x over previous
import numpy as np
import jax
import jax.numpy as jnp
from jax import lax
from jax.experimental import pallas as pl
from jax.experimental.pallas import tpu as pltpu
from jax.experimental.pallas import tpu_sc as plsc

N_NODES = 10000
N_EDGES = 160000
K = 16
HID = 128
BLK = 1000

NT = 16
EPT = 10080
E_PAD = NT * EPT
CH = 80
NCH = EPT // CH
NBATCH = NCH // 2
RPT = 624
TAIL = N_NODES - NT * RPT
TRASH = N_NODES


NB = 2


def _sc_body(feats, srcs, dsts, tcb, zeros, accs_out,
             idx2, dstq, tq, basq, rows0, rows1,
             acc_sh, semt, semg0, semg1, semsc0, semsc1, semd):
    c = lax.axis_index("c")
    s = lax.axis_index("s")
    f32 = jnp.float32
    rowss = (rows0, rows1)
    semgs = (semg0, semg1)
    semscs = (semsc0, semsc1)

    pltpu.sync_copy(srcs.at[c * NT + s], idx2)

    def _zero_acc():
        pltpu.sync_copy(zeros.at[pl.ds(0, RPT), :],
                        acc_sh.at[pl.ds(s * RPT, RPT), :])

        @pl.when(s == 0)
        def _():
            pltpu.sync_copy(zeros.at[pl.ds(0, TAIL), :],
                            acc_sh.at[pl.ds(NT * RPT, TAIL), :])

    def _writeback(task):
        pltpu.sync_copy(acc_sh.at[pl.ds(s * RPT, RPT), :],
                        accs_out.at[task, pl.ds(s * RPT, RPT), :])

        @pl.when(s == 0)
        def _():
            pltpu.sync_copy(acc_sh.at[pl.ds(NT * RPT, TAIL), :],
                            accs_out.at[task, pl.ds(NT * RPT, TAIL), :])

    def _basis(p, tb, ba):
        for g in range(2 * CH // 16):
            b = jnp.ones((16,), f32)
            for d in range(4):
                bf = ((p >> d) & 1).astype(f32)
                td = tb[d, pl.ds(g * 16, 16)]
                b = b * ((1.0 - td) + bf * (2.0 * td - 1.0))
            ba[pl.ds(g * 16, 16)] = b

    def _scale(ba, boff, rw):
        @plsc.parallel_loop(0, CH, step=1, unroll=4)
        def scale(e):
            bv = ba[pl.ds(boff + e, 16)][0]
            for u in range(8):
                sl = pl.ds(u * 16, 16)
                rw[e, sl] = rw[e, sl] * bv

    def _drain_scatter(b):
        pltpu.make_async_copy(zeros.at[pl.ds(0, CH), :], rowss[b],
                              semscs[b]).wait()

    def one_pass(p, _):
        _zero_acc()
        plsc.subcore_barrier()

        def batch(q, _):
            dgs = []
            for b in range(NB):
                @pl.when(q > 0)
                def _():
                    _drain_scatter(b)
                dgs.append(pltpu.async_copy(feats.at[idx2.at[q * NB + b]],
                                            rowss[b], semgs[b]))
            dt = pltpu.async_copy(tcb.at[s, q], tq, semt)
            dd = pltpu.async_copy(dstq_hbm_slice(q), dstq, semd)
            dt.wait()
            _basis(p, tq, basq)
            dd.wait()
            for b in range(NB):
                dgs[b].wait()
                _scale(basq, b * CH, rowss[b])
                pltpu.async_copy(rowss[b], acc_sh.at[dstq.at[b]], semscs[b],
                                 add=True)
            return 0

        def dstq_hbm_slice(q):
            return dsts.at[s, q]
        lax.fori_loop(0, NBATCH, batch, 0)
        for b in range(NB):
            _drain_scatter(b)

        plsc.subcore_barrier()
        _writeback(c * K + p)
        return 0

    lax.fori_loop(0, K, one_pass, 0)

    _zero_acc()
    plsc.subcore_barrier()

    def _fill_ones(r, _):
        for u in range(8):
            rows0[r, pl.ds(u * 16, 16)] = jnp.ones((16,), f32)
        return 0
    lax.fori_loop(0, CH, _fill_ones, 0)

    def dchunk(q, _):
        pltpu.async_copy(dsts.at[s, q], dstq, semd).wait()
        for b in range(NB):
            pltpu.sync_copy(rows0, acc_sh.at[dstq.at[b]], add=True)
        return 0
    lax.fori_loop(0, NBATCH, dchunk, 0)
    plsc.subcore_barrier()

    @pl.when(c == 0)
    def _():
        _writeback(2 * K)


_sc_scatter = pl.kernel(
    _sc_body,
    mesh=plsc.VectorSubcoreMesh(core_axis_name="c", subcore_axis_name="s"),
    out_type=jax.ShapeDtypeStruct((2 * K + 1, N_NODES, HID), jnp.float32),
    scratch_types=[
        pltpu.VMEM((NCH, CH), jnp.int32),
        pltpu.VMEM((NB, CH), jnp.int32),
        pltpu.VMEM((4, NB * CH), jnp.float32),
        pltpu.VMEM((NB * CH + 16,), jnp.float32),
        pltpu.VMEM((CH, HID), jnp.float32),
        pltpu.VMEM((CH, HID), jnp.float32),
        pltpu.VMEM_SHARED((N_NODES + 8, HID), jnp.float32),
        pltpu.SemaphoreType.DMA,
        pltpu.SemaphoreType.DMA,
        pltpu.SemaphoreType.DMA,
        pltpu.SemaphoreType.DMA,
        pltpu.SemaphoreType.DMA,
        pltpu.SemaphoreType.DMA,
    ],
)


def _dense_body(accx_ref, acch_ref, deg_ref, x_ref, h_ref,
                wx_ref, wh_ref, rx_ref, rh_ref, bx_ref, bh_ref, out_ref):
    f32 = jnp.float32
    dn = (((1,), (0,)), ((), ()))
    ax = lax.dot_general(x_ref[...], rx_ref[...], dn, preferred_element_type=f32)
    ah = lax.dot_general(h_ref[...], rh_ref[...], dn, preferred_element_type=f32)
    axs = jnp.zeros_like(ax)
    ahs = jnp.zeros_like(ah)
    for k in range(K):
        axs += lax.dot_general(accx_ref[k], wx_ref[k], dn, preferred_element_type=f32)
        ahs += lax.dot_general(acch_ref[k], wh_ref[k], dn, preferred_element_type=f32)
    dinv = 1.0 / jnp.maximum(deg_ref[0][:, 0:1], 1.0)
    ax = ax + axs * dinv + bx_ref[...]
    ah = ah + ahs * dinv + bh_ref[...]
    xr_o = ax[:, 0:128]
    xz_o = ax[:, 128:256]
    xn_o = ax[:, 256:384]
    hr_o = ah[:, 0:128]
    hz_o = ah[:, 128:256]
    r = jax.nn.sigmoid(xr_o + hr_o)
    z = jax.nn.sigmoid(xz_o + hz_o)
    n = jnp.tanh(xn_o + r * hr_o)
    out_ref[...] = (1.0 - z) * n + z * h_ref[...]


def _dense_phase(accs, x, hidden, wx, wh, rx, rh, bx, bh):
    grid = (N_NODES // BLK,)
    return pl.pallas_call(
        _dense_body,
        grid=grid,
        in_specs=[
            pl.BlockSpec((K, BLK, HID), lambda i: (0, i, 0)),
            pl.BlockSpec((K, BLK, HID), lambda i: (1, i, 0)),
            pl.BlockSpec((1, BLK, HID), lambda i: (2 * K, i, 0)),
            pl.BlockSpec((BLK, HID), lambda i: (i, 0)),
            pl.BlockSpec((BLK, HID), lambda i: (i, 0)),
            pl.BlockSpec((K, HID, 3 * HID), lambda i: (0, 0, 0)),
            pl.BlockSpec((K, HID, 2 * HID), lambda i: (0, 0, 0)),
            pl.BlockSpec((HID, 3 * HID), lambda i: (0, 0)),
            pl.BlockSpec((HID, 2 * HID), lambda i: (0, 0)),
            pl.BlockSpec((1, 3 * HID), lambda i: (0, 0)),
            pl.BlockSpec((1, 2 * HID), lambda i: (0, 0)),
        ],
        out_specs=pl.BlockSpec((BLK, HID), lambda i: (i, 0)),
        out_shape=jax.ShapeDtypeStruct((N_NODES, HID), jnp.float32),
    )(accs, accs, accs, x, hidden, wx, wh, rx, rh, bx, bh)


def kernel(x, hidden, edge_index, edge_attr,
           xr_w, xr_root, xr_b, hr_w, hr_root, hr_b,
           xz_w, xz_root, xz_b, hz_w, hz_root, hz_b,
           xn_w, xn_root, xn_b):
    src = edge_index[0].astype(jnp.int32)
    dst = edge_index[1].astype(jnp.int32)
    npad = E_PAD - N_EDGES
    srcp = jnp.concatenate([src, jnp.zeros((npad,), jnp.int32)])
    dstp = jnp.concatenate([dst, jnp.full((npad,), TRASH, jnp.int32)])
    eatp = jnp.concatenate(
        [edge_attr, jnp.full((npad, 4), 0.5, jnp.float32)])
    feats = jnp.concatenate([x, hidden], axis=0)
    srcs = jnp.stack([srcp, srcp + N_NODES]).reshape(2 * NT, NCH, CH)
    dsts = dstp.reshape(NT, NBATCH, 2, CH)
    tcb = eatp.reshape(NT, NBATCH, 2 * CH, 4).transpose(0, 1, 3, 2)
    zeros = jnp.zeros((RPT + TAIL + 8, HID), jnp.float32)

    accs = _sc_scatter(feats, srcs, dsts, tcb, zeros)

    wx = jnp.concatenate([xr_w, xz_w, xn_w], axis=2)
    wh = jnp.concatenate([hr_w, hz_w], axis=2)
    rx = jnp.concatenate([xr_root, xz_root, xn_root], axis=1)
    rh = jnp.concatenate([hr_root, hz_root], axis=1)
    bx = jnp.concatenate([xr_b, xz_b, xn_b])[None, :]
    bh = jnp.concatenate([hr_b, hz_b])[None, :]
    return _dense_phase(accs, x, hidden, wx, wh, rx, rh, bx, bh)


# speedup vs baseline: 2.3725x; 1.4007x over previous
import numpy as np
import jax
import jax.numpy as jnp
from jax import lax
from jax.experimental import pallas as pl
from jax.experimental.pallas import tpu as pltpu
from jax.experimental.pallas import tpu_sc as plsc

N_NODES = 10000
N_EDGES = 160000
K = 16
HID = 128
BLK = 1000

NT = 16
EPT = N_EDGES // NT
CH = 80
NCH = EPT // CH
RPT = 624
TAIL = N_NODES - NT * RPT


NB = 2


def _sc_body(feats, srcs, dsts, tcb, zeros, accs_out,
             idx2, dstb0, dstb1, tbuf0, tbuf1, bas0, bas1, rows0, rows1,
             acc_sh, semt0, semt1, semg0, semg1, semsc0, semsc1,
             semd0, semd1):
    c = lax.axis_index("c")
    s = lax.axis_index("s")
    f32 = jnp.float32
    tbufs = (tbuf0, tbuf1)
    bass = (bas0, bas1)
    rowss = (rows0, rows1)
    dstbs = (dstb0, dstb1)
    semts = (semt0, semt1)
    semgs = (semg0, semg1)
    semscs = (semsc0, semsc1)
    semds = (semd0, semd1)

    pltpu.sync_copy(srcs.at[c * NT + s], idx2)

    def _zero_acc():
        pltpu.sync_copy(zeros.at[pl.ds(0, RPT), :],
                        acc_sh.at[pl.ds(s * RPT, RPT), :])

        @pl.when(s == 0)
        def _():
            pltpu.sync_copy(zeros.at[pl.ds(0, TAIL), :],
                            acc_sh.at[pl.ds(NT * RPT, TAIL), :])

    def _writeback(task):
        pltpu.sync_copy(acc_sh.at[pl.ds(s * RPT, RPT), :],
                        accs_out.at[task, pl.ds(s * RPT, RPT), :])

        @pl.when(s == 0)
        def _():
            pltpu.sync_copy(acc_sh.at[pl.ds(NT * RPT, TAIL), :],
                            accs_out.at[task, pl.ds(NT * RPT, TAIL), :])

    def _basis(p, tb, ba):
        for g in range(CH // 16):
            b = jnp.ones((16,), f32)
            for d in range(4):
                bf = ((p >> d) & 1).astype(f32)
                td = tb[d, pl.ds(g * 16, 16)]
                b = b * ((1.0 - td) + bf * (2.0 * td - 1.0))
            ba[pl.ds(g * 16, 16)] = b

    def _scale(ba, rw):
        @plsc.parallel_loop(0, CH, step=1, unroll=4)
        def scale(e):
            bv = ba[pl.ds(e, 16)][0]
            for u in range(8):
                sl = pl.ds(u * 16, 16)
                rw[e, sl] = rw[e, sl] * bv

    def _drain_scatter(b):
        pltpu.make_async_copy(zeros.at[pl.ds(0, CH), :], rowss[b],
                              semscs[b]).wait()

    def one_pass(p, _):
        _zero_acc()
        plsc.subcore_barrier()

        def batch(q, _):
            dts, dgs, dds = [], [], []
            for b in range(NB):
                j = q * NB + b

                @pl.when(q > 0)
                def _():
                    _drain_scatter(b)
                dts.append(pltpu.async_copy(tcb.at[s, j], tbufs[b], semts[b]))
                dds.append(pltpu.async_copy(
                    dsts.at[pl.ds(s * EPT + j * CH, CH)], dstbs[b], semds[b]))
                dgs.append(pltpu.async_copy(feats.at[idx2.at[j]], rowss[b],
                                            semgs[b]))
            for b in range(NB):
                j = q * NB + b
                dts[b].wait()
                _basis(p, tbufs[b], bass[b])
                dgs[b].wait()
                _scale(bass[b], rowss[b])
                dds[b].wait()
                pltpu.async_copy(rowss[b], acc_sh.at[dstbs[b]], semscs[b],
                                 add=True)
            return 0
        lax.fori_loop(0, NCH // NB, batch, 0)
        for b in range(NB):
            _drain_scatter(b)

        jt = (NCH // NB) * NB
        pltpu.async_copy(tcb.at[s, jt], tbufs[0], semts[0]).wait()
        _basis(p, tbufs[0], bass[0])
        pltpu.async_copy(feats.at[idx2.at[jt]], rowss[0], semgs[0]).wait()
        _scale(bass[0], rowss[0])
        pltpu.async_copy(dsts.at[pl.ds(s * EPT + jt * CH, CH)], dstb0,
                         semd0).wait()
        pltpu.async_copy(rowss[0], acc_sh.at[dstb0], semscs[0],
                         add=True)
        _drain_scatter(0)

        plsc.subcore_barrier()
        _writeback(c * K + p)
        return 0

    lax.fori_loop(0, K, one_pass, 0)

    _zero_acc()
    plsc.subcore_barrier()

    def _fill_ones(r, _):
        for u in range(8):
            rows0[r, pl.ds(u * 16, 16)] = jnp.ones((16,), f32)
        return 0
    lax.fori_loop(0, CH, _fill_ones, 0)

    def dchunk(j, _):
        pltpu.async_copy(dsts.at[pl.ds(s * EPT + j * CH, CH)], dstb0,
                         semd0).wait()
        pltpu.sync_copy(rows0, acc_sh.at[dstb0], add=True)
        return 0
    lax.fori_loop(0, NCH, dchunk, 0)
    plsc.subcore_barrier()

    @pl.when(c == 0)
    def _():
        _writeback(2 * K)


_sc_scatter = pl.kernel(
    _sc_body,
    mesh=plsc.VectorSubcoreMesh(core_axis_name="c", subcore_axis_name="s"),
    out_type=jax.ShapeDtypeStruct((2 * K + 1, N_NODES, HID), jnp.float32),
    scratch_types=[
        pltpu.VMEM((NCH, CH), jnp.int32),
        pltpu.VMEM((CH,), jnp.int32),
        pltpu.VMEM((CH,), jnp.int32),
        pltpu.VMEM((4, CH), jnp.float32),
        pltpu.VMEM((4, CH), jnp.float32),
        pltpu.VMEM((CH + 16,), jnp.float32),
        pltpu.VMEM((CH + 16,), jnp.float32),
        pltpu.VMEM((CH, HID), jnp.float32),
        pltpu.VMEM((CH, HID), jnp.float32),
        pltpu.VMEM_SHARED((N_NODES, HID), jnp.float32),
        pltpu.SemaphoreType.DMA,
        pltpu.SemaphoreType.DMA,
        pltpu.SemaphoreType.DMA,
        pltpu.SemaphoreType.DMA,
        pltpu.SemaphoreType.DMA,
        pltpu.SemaphoreType.DMA,
        pltpu.SemaphoreType.DMA,
        pltpu.SemaphoreType.DMA,
    ],
)


def _dense_body(accx_ref, acch_ref, deg_ref, x_ref, h_ref,
                wx_ref, wh_ref, rx_ref, rh_ref, bx_ref, bh_ref, out_ref):
    f32 = jnp.float32
    dn = (((1,), (0,)), ((), ()))
    ax = lax.dot_general(x_ref[...], rx_ref[...], dn, preferred_element_type=f32)
    ah = lax.dot_general(h_ref[...], rh_ref[...], dn, preferred_element_type=f32)
    axs = jnp.zeros_like(ax)
    ahs = jnp.zeros_like(ah)
    for k in range(K):
        axs += lax.dot_general(accx_ref[k], wx_ref[k], dn, preferred_element_type=f32)
        ahs += lax.dot_general(acch_ref[k], wh_ref[k], dn, preferred_element_type=f32)
    dinv = 1.0 / jnp.maximum(deg_ref[0][:, 0:1], 1.0)
    ax = ax + axs * dinv + bx_ref[...]
    ah = ah + ahs * dinv + bh_ref[...]
    xr_o = ax[:, 0:128]
    xz_o = ax[:, 128:256]
    xn_o = ax[:, 256:384]
    hr_o = ah[:, 0:128]
    hz_o = ah[:, 128:256]
    r = jax.nn.sigmoid(xr_o + hr_o)
    z = jax.nn.sigmoid(xz_o + hz_o)
    n = jnp.tanh(xn_o + r * hr_o)
    out_ref[...] = (1.0 - z) * n + z * h_ref[...]


def _dense_phase(accs, x, hidden, wx, wh, rx, rh, bx, bh):
    grid = (N_NODES // BLK,)
    return pl.pallas_call(
        _dense_body,
        grid=grid,
        in_specs=[
            pl.BlockSpec((K, BLK, HID), lambda i: (0, i, 0)),
            pl.BlockSpec((K, BLK, HID), lambda i: (1, i, 0)),
            pl.BlockSpec((1, BLK, HID), lambda i: (2 * K, i, 0)),
            pl.BlockSpec((BLK, HID), lambda i: (i, 0)),
            pl.BlockSpec((BLK, HID), lambda i: (i, 0)),
            pl.BlockSpec((K, HID, 3 * HID), lambda i: (0, 0, 0)),
            pl.BlockSpec((K, HID, 2 * HID), lambda i: (0, 0, 0)),
            pl.BlockSpec((HID, 3 * HID), lambda i: (0, 0)),
            pl.BlockSpec((HID, 2 * HID), lambda i: (0, 0)),
            pl.BlockSpec((1, 3 * HID), lambda i: (0, 0)),
            pl.BlockSpec((1, 2 * HID), lambda i: (0, 0)),
        ],
        out_specs=pl.BlockSpec((BLK, HID), lambda i: (i, 0)),
        out_shape=jax.ShapeDtypeStruct((N_NODES, HID), jnp.float32),
    )(accs, accs, accs, x, hidden, wx, wh, rx, rh, bx, bh)


def kernel(x, hidden, edge_index, edge_attr,
           xr_w, xr_root, xr_b, hr_w, hr_root, hr_b,
           xz_w, xz_root, xz_b, hz_w, hz_root, hz_b,
           xn_w, xn_root, xn_b):
    src = edge_index[0].astype(jnp.int32)
    dst = edge_index[1].astype(jnp.int32)
    feats = jnp.concatenate([x, hidden], axis=0)
    srcs = jnp.stack([src, src + N_NODES]).reshape(2 * NT, NCH, CH)
    dsts = dst
    tcb = edge_attr.reshape(NT, NCH, CH, 4).transpose(0, 1, 3, 2)
    zeros = jnp.zeros((RPT + TAIL, HID), jnp.float32)

    accs = _sc_scatter(feats, srcs, dsts, tcb, zeros)

    wx = jnp.concatenate([xr_w, xz_w, xn_w], axis=2)
    wh = jnp.concatenate([hr_w, hz_w], axis=2)
    rx = jnp.concatenate([xr_root, xz_root, xn_root], axis=1)
    rh = jnp.concatenate([hr_root, hz_root], axis=1)
    bx = jnp.concatenate([xr_b, xz_b, xn_b])[None, :]
    bh = jnp.concatenate([hr_b, hz_b])[None, :]
    return _dense_phase(accs, x, hidden, wx, wh, rx, rh, bx, bh)
